# baseline (speedup 1.0000x reference)
import jax
import jax.numpy as jnp
from jax.experimental import pallas as pl

NUM_EXPERTS = 8
IN_FEATURES = 768
OUT_FEATURES = 768
N_TOKENS = 8192

BT = 512


def _moe_block(xb_ref, wall_ref, be_ref, wgt_ref, bg_ref, out_ref):
    logits = jnp.dot(xb_ref[...], wgt_ref[...], preferred_element_type=jnp.float32)
    logits = logits + bg_ref[...]
    m = jnp.max(logits, axis=1, keepdims=True)
    eg = jnp.exp(logits - m)
    gate = eg / jnp.sum(eg, axis=1, keepdims=True)

    y = jnp.dot(xb_ref[...], wall_ref[...], preferred_element_type=jnp.float32)

    acc = jnp.dot(gate, be_ref[...], preferred_element_type=jnp.float32)
    for e in range(NUM_EXPERTS):
        acc = acc + gate[:, e:e + 1] * y[:, e * OUT_FEATURES:(e + 1) * OUT_FEATURES]
    out_ref[...] = acc


def kernel(x, We, be, Wg, bg):
    wall = jnp.transpose(We, (2, 0, 1)).reshape(
        IN_FEATURES, NUM_EXPERTS * OUT_FEATURES).astype(jnp.bfloat16)
    xb = x.astype(jnp.bfloat16)
    wgt = Wg.T.astype(jnp.bfloat16)
    bg2 = bg.reshape(1, NUM_EXPERTS)

    grid = (N_TOKENS // BT,)
    return pl.pallas_call(
        _moe_block,
        grid=grid,
        in_specs=[
            pl.BlockSpec((BT, IN_FEATURES), lambda i: (i, 0)),
            pl.BlockSpec((IN_FEATURES, NUM_EXPERTS * OUT_FEATURES), lambda i: (0, 0)),
            pl.BlockSpec((NUM_EXPERTS, OUT_FEATURES), lambda i: (0, 0)),
            pl.BlockSpec((IN_FEATURES, NUM_EXPERTS), lambda i: (0, 0)),
            pl.BlockSpec((1, NUM_EXPERTS), lambda i: (0, 0)),
        ],
        out_specs=pl.BlockSpec((BT, OUT_FEATURES), lambda i: (i, 0)),
        out_shape=jax.ShapeDtypeStruct((N_TOKENS, OUT_FEATURES), jnp.float32),
    )(xb, wall, be, wgt, bg2)


# speedup vs baseline: 1.1090x; 1.1090x over previous
import jax
import jax.numpy as jnp
from jax.experimental import pallas as pl

NUM_EXPERTS = 8
IN_FEATURES = 768
OUT_FEATURES = 768
N_TOKENS = 8192

BT = 512


def _moe_block(x_ref, wall_ref, be_ref, wgt_ref, bg_ref, out_ref):
    logits = jnp.dot(x_ref[...], wgt_ref[...], preferred_element_type=jnp.float32)
    logits = logits + bg_ref[...]
    m = jnp.max(logits, axis=1, keepdims=True)
    eg = jnp.exp(logits - m)
    gate = eg / jnp.sum(eg, axis=1, keepdims=True)

    y = jnp.dot(x_ref[...], wall_ref[...], preferred_element_type=jnp.float32)

    acc = jnp.dot(gate, be_ref[...], preferred_element_type=jnp.float32)
    for e in range(NUM_EXPERTS):
        acc = acc + gate[:, e:e + 1] * y[:, e * OUT_FEATURES:(e + 1) * OUT_FEATURES]
    out_ref[...] = acc


def kernel(x, We, be, Wg, bg):
    wall = jnp.transpose(We, (2, 0, 1)).reshape(
        IN_FEATURES, NUM_EXPERTS * OUT_FEATURES)
    wgt = Wg.T
    bg2 = bg.reshape(1, NUM_EXPERTS)

    grid = (N_TOKENS // BT,)
    return pl.pallas_call(
        _moe_block,
        grid=grid,
        in_specs=[
            pl.BlockSpec((BT, IN_FEATURES), lambda i: (i, 0)),
            pl.BlockSpec((IN_FEATURES, NUM_EXPERTS * OUT_FEATURES), lambda i: (0, 0)),
            pl.BlockSpec((NUM_EXPERTS, OUT_FEATURES), lambda i: (0, 0)),
            pl.BlockSpec((IN_FEATURES, NUM_EXPERTS), lambda i: (0, 0)),
            pl.BlockSpec((1, NUM_EXPERTS), lambda i: (0, 0)),
        ],
        out_specs=pl.BlockSpec((BT, OUT_FEATURES), lambda i: (i, 0)),
        out_shape=jax.ShapeDtypeStruct((N_TOKENS, OUT_FEATURES), jnp.float32),
    )(x, wall, be, wgt, bg2)
